# Initial kernel scaffold; baseline (speedup 1.0000x reference)
import jax
import jax.numpy as jnp
from jax.experimental import pallas as pl


def kernel(node_attrs, positions, edge_attrs, edge_index, params):
    raise NotImplementedError("write your pallas kernel here")


# speedup vs baseline: 1.2568x; 1.2568x over previous
import functools

import jax
import jax.numpy as jnp
from jax import lax
from jax.experimental import pallas as pl
from jax.experimental.pallas import tpu as pltpu
from jax.experimental.pallas import tpu_sc as plsc

_N = 10000
_E = 160000
_C = 256
_NUM_RBF = 16
_CUTOFF = 5.0
_NUM_BLOCKS = 5

_NW = 32
_EPW = 5120
_E_PAD = _NW * _EPW
_N_PAD = 10240
_NHALF = 5120
_GCHUNK = 128
_SCHUNK = 512

_ETILE = 2048
_NTILE = 512


def _silu(x):
    return x * jax.nn.sigmoid(x)


def _row_spec(tile, width):
    return pl.BlockSpec((tile, width), lambda i: (i, 0))


def _full_spec(shape):
    return pl.BlockSpec(shape, lambda i: tuple(0 for _ in shape))


def _embed_nodes_body(x_ref, w1_ref, b1_ref, w2_ref, b2_ref, ws_ref, wd_ref,
                      h_ref, ps_ref, pd_ref):
    t = _silu(x_ref[...] @ w1_ref[...] + b1_ref[...])
    h = t @ w2_ref[...] + b2_ref[...]
    h_ref[...] = h
    ps_ref[...] = h @ ws_ref[...]
    pd_ref[...] = h @ wd_ref[...]


def _embed_nodes(x, w1, b1, w2, b2, ws, wd):
    n = x.shape[0]
    grid = (pl.cdiv(n, _NTILE),)
    return pl.pallas_call(
        _embed_nodes_body,
        grid=grid,
        in_specs=[
            _row_spec(_NTILE, x.shape[1]),
            _full_spec(w1.shape), _full_spec(b1.shape),
            _full_spec(w2.shape), _full_spec(b2.shape),
            _full_spec(ws.shape), _full_spec(wd.shape),
        ],
        out_specs=[
            _row_spec(_NTILE, _C), _row_spec(_NTILE, _C), _row_spec(_NTILE, _C),
        ],
        out_shape=[
            jax.ShapeDtypeStruct((n, _C), jnp.float32),
            jax.ShapeDtypeStruct((n, _C), jnp.float32),
            jax.ShapeDtypeStruct((n, _C), jnp.float32),
        ],
    )(x, w1, b1, w2, b2, ws, wd)


def _embed_edges_body(x_ref, w1_ref, b1_ref, w2_ref, b2_ref, out_ref):
    t = _silu(x_ref[...] @ w1_ref[...] + b1_ref[...])
    out_ref[...] = t @ w2_ref[...] + b2_ref[...]


def _embed_edges(x, w1, b1, w2, b2):
    n = x.shape[0]
    grid = (pl.cdiv(n, _ETILE),)
    return pl.pallas_call(
        _embed_edges_body,
        grid=grid,
        in_specs=[
            _row_spec(_ETILE, x.shape[1]),
            _full_spec(w1.shape), _full_spec(b1.shape),
            _full_spec(w2.shape), _full_spec(b2.shape),
        ],
        out_specs=_row_spec(_ETILE, _C),
        out_shape=jax.ShapeDtypeStruct((n, _C), jnp.float32),
    )(x, w1, b1, w2, b2)


def _edge_body(asrc_ref, adst_ref, ef_ref, psrc_ref, pdst_ref,
               we_ref, wr_ref, b1_ref, w2_ref, b2_ref,
               wx1_ref, bx1_ref, wx2_ref, bx2_ref,
               m_ref, wv_ref):
    rel = psrc_ref[...] - pdst_ref[...]
    d2 = jnp.sum(rel * rel, axis=-1, keepdims=True)
    dist = jnp.sqrt(d2 + 1e-8)
    nvec = (lax.broadcasted_iota(jnp.int32, (1, _NUM_RBF), 1) + 1
            ).astype(jnp.float32)
    rbf = (jnp.sqrt(2.0 / _CUTOFF) * jnp.sin(nvec * (jnp.pi / _CUTOFF) * dist)
           / (dist + 1e-8))
    pre = (asrc_ref[...] + adst_ref[...] + ef_ref[...] @ we_ref[...]
           + rbf @ wr_ref[...] + b1_ref[...])
    m = _silu(pre) @ w2_ref[...] + b2_ref[...]
    m_ref[...] = m
    t2 = _silu(m @ wx1_ref[...] + bx1_ref[...])
    xw = t2 @ wx2_ref[...] + bx2_ref[...]
    wv_ref[...] = rel / (dist + 1.0) * xw


def _edge_mlp(asrc, adst, ef, psrc, pdst, we, wr, b1, w2, b2, wx1, bx1,
              wx2r, bx2r):
    n = asrc.shape[0]
    grid = (pl.cdiv(n, _ETILE),)
    return pl.pallas_call(
        _edge_body,
        grid=grid,
        in_specs=[
            _row_spec(_ETILE, _C), _row_spec(_ETILE, _C), _row_spec(_ETILE, _C),
            _row_spec(_ETILE, 8), _row_spec(_ETILE, 8),
            _full_spec(we.shape), _full_spec(wr.shape), _full_spec(b1.shape),
            _full_spec(w2.shape), _full_spec(b2.shape),
            _full_spec(wx1.shape), _full_spec(bx1.shape),
            _full_spec(wx2r.shape), _full_spec(bx2r.shape),
        ],
        out_specs=[_row_spec(_ETILE, _C), _row_spec(_ETILE, 8)],
        out_shape=[
            jax.ShapeDtypeStruct((n, _C), jnp.float32),
            jax.ShapeDtypeStruct((n, 8), jnp.float32),
        ],
    )(asrc, adst, ef, psrc, pdst, we, wr, b1, w2, b2, wx1, bx1, wx2r, bx2r)


def _node_body(h_ref, agg_ref, pos_ref, dp0_ref, dp1_ref,
               w1h_ref, w1a_ref, b1_ref, w2_ref, b2_ref, g_ref, bt_ref,
               ws_ref, wd_ref,
               h_out_ref, pos_out_ref, ps_ref, pd_ref):
    h = h_ref[...]
    t = _silu(h @ w1h_ref[...] + agg_ref[...] @ w1a_ref[...] + b1_ref[...])
    d = t @ w2_ref[...] + b2_ref[...]
    hn = g_ref[...] * (h + d) + bt_ref[...]
    h_out_ref[...] = hn
    pos_out_ref[...] = pos_ref[...] + dp0_ref[...] + dp1_ref[...]
    if ps_ref is not None:
        ps_ref[...] = hn @ ws_ref[...]
        pd_ref[...] = hn @ wd_ref[...]


def _node_update(h, agg, pos, dp0, dp1, w1h, w1a, b1, w2, b2, gamma, beta,
                 ws=None, wd=None):
    n = h.shape[0]
    grid = (pl.cdiv(n, _NTILE),)
    has_proj = ws is not None
    if not has_proj:
        ws = jnp.zeros((8, _C), jnp.float32)
        wd = jnp.zeros((8, _C), jnp.float32)
    out_specs = [_row_spec(_NTILE, _C), _row_spec(_NTILE, 8)]
    out_shape = [jax.ShapeDtypeStruct((n, _C), jnp.float32),
                 jax.ShapeDtypeStruct((n, 8), jnp.float32)]
    if has_proj:
        out_specs += [_row_spec(_NTILE, _C), _row_spec(_NTILE, _C)]
        out_shape += [jax.ShapeDtypeStruct((n, _C), jnp.float32),
                      jax.ShapeDtypeStruct((n, _C), jnp.float32)]
        body = _node_body
    else:
        def body(h_ref, agg_ref, pos_ref, dp0_ref, dp1_ref, w1h_ref, w1a_ref,
                 b1_ref, w2_ref, b2_ref, g_ref, bt_ref, ws_ref, wd_ref,
                 h_out_ref, pos_out_ref):
            _node_body(h_ref, agg_ref, pos_ref, dp0_ref, dp1_ref, w1h_ref,
                       w1a_ref, b1_ref, w2_ref, b2_ref, g_ref, bt_ref,
                       ws_ref, wd_ref, h_out_ref, pos_out_ref, None, None)
    res = pl.pallas_call(
        body,
        grid=grid,
        in_specs=[
            _row_spec(_NTILE, _C), _row_spec(_NTILE, _C), _row_spec(_NTILE, 8),
            _row_spec(_NTILE, 8), _row_spec(_NTILE, 8),
            _full_spec(w1h.shape), _full_spec(w1a.shape), _full_spec(b1.shape),
            _full_spec(w2.shape), _full_spec(b2.shape),
            _full_spec(gamma.shape), _full_spec(beta.shape),
            _full_spec(ws.shape), _full_spec(wd.shape),
        ],
        out_specs=out_specs,
        out_shape=out_shape,
    )(h, agg, pos, dp0, dp1, w1h, w1a, b1, w2, b2, gamma, beta, ws, wd)
    if has_proj:
        return res
    return res[0], res[1], None, None


@functools.cache
def _sc_mesh():
    return plsc.VectorSubcoreMesh(core_axis_name="c", subcore_axis_name="s")


_SC_LINEAR = pltpu.CompilerParams(use_tc_tiling_on_sc=False)


def _gather_feat_kernel(ps_hbm, pd_hbm, src_hbm, dst_hbm,
                        asrc_hbm, adst_hbm,
                        idxs_v, idxd_v, rows_v, sem):
    wid = lax.axis_index("s") * 2 + lax.axis_index("c")
    base = wid * _EPW
    pltpu.sync_copy(src_hbm.at[pl.ds(base, _EPW)], idxs_v)
    pltpu.sync_copy(dst_hbm.at[pl.ds(base, _EPW)], idxd_v)

    nchunk = _EPW // _GCHUNK

    def body_s(k, _):
        off = k * _GCHUNK
        pltpu.async_copy(
            ps_hbm.at[idxs_v.at[pl.ds(off, _GCHUNK)]], rows_v, sem).wait()
        pltpu.sync_copy(rows_v, asrc_hbm.at[pl.ds(base + off, _GCHUNK)])
        return _

    lax.fori_loop(0, nchunk, body_s, 0)

    def body_d(k, _):
        off = k * _GCHUNK
        pltpu.async_copy(
            pd_hbm.at[idxd_v.at[pl.ds(off, _GCHUNK)]], rows_v, sem).wait()
        pltpu.sync_copy(rows_v, adst_hbm.at[pl.ds(base + off, _GCHUNK)])
        return _

    lax.fori_loop(0, nchunk, body_d, 0)


def _sc_gather_feat(ps, pd, src, dst):
    kfn = pl.kernel(
        _gather_feat_kernel,
        out_type=[
            jax.ShapeDtypeStruct((_E_PAD, _C), jnp.float32),
            jax.ShapeDtypeStruct((_E_PAD, _C), jnp.float32),
        ],
        mesh=_sc_mesh(),
        scratch_types=[
            pltpu.VMEM((_EPW,), jnp.int32),
            pltpu.VMEM((_EPW,), jnp.int32),
            pltpu.VMEM((_GCHUNK, _C), jnp.float32),
            pltpu.SemaphoreType.DMA,
        ],
    )
    return kfn(ps, pd, src, dst)


def _gather_pos_kernel(pos_hbm, src_hbm, dst_hbm, psrc_hbm, pdst_hbm,
                       idx_v, posbuf_v, sem):
    wid = lax.axis_index("s") * 2 + lax.axis_index("c")
    base = wid * _EPW
    pltpu.sync_copy(src_hbm.at[pl.ds(base, _EPW)], idx_v)
    pltpu.async_copy(pos_hbm.at[idx_v], posbuf_v, sem).wait()
    pltpu.sync_copy(posbuf_v, psrc_hbm.at[pl.ds(base, _EPW)])
    pltpu.sync_copy(dst_hbm.at[pl.ds(base, _EPW)], idx_v)
    pltpu.async_copy(pos_hbm.at[idx_v], posbuf_v, sem).wait()
    pltpu.sync_copy(posbuf_v, pdst_hbm.at[pl.ds(base, _EPW)])


def _sc_gather_pos(pos_pad, src, dst):
    kfn = pl.kernel(
        _gather_pos_kernel,
        out_type=[
            jax.ShapeDtypeStruct((_E_PAD, 8), jnp.float32),
            jax.ShapeDtypeStruct((_E_PAD, 8), jnp.float32),
        ],
        mesh=_sc_mesh(),
        scratch_types=[
            pltpu.VMEM((_EPW,), jnp.int32),
            pltpu.VMEM((_EPW, 8), jnp.float32),
            pltpu.SemaphoreType.DMA,
        ],
        compiler_params=_SC_LINEAR,
    )
    return kfn(pos_pad, src, dst)


def _scatter_m_kernel(m_hbm, dst0_hbm, dst1_hbm, zero_c_hbm, agg_hbm,
                      idx_v, mbuf_v, acc_sh, sem):
    cid = lax.axis_index("c")
    sid = lax.axis_index("s")
    nrows = _NHALF // 16
    epc = _E_PAD // 16
    nchunk = epc // _SCHUNK

    for p, dst_hbm in enumerate((dst0_hbm, dst1_hbm)):
        pltpu.sync_copy(zero_c_hbm, acc_sh.at[pl.ds(sid * nrows, nrows)])
        plsc.subcore_barrier()

        def body(k, _, dst_hbm=dst_hbm):
            e0 = sid * epc + k * _SCHUNK
            pltpu.sync_copy(dst_hbm.at[pl.ds(e0, _SCHUNK)], idx_v)
            pltpu.sync_copy(
                m_hbm.at[pl.ds(e0, _SCHUNK), pl.ds(cid * 128, 128)], mbuf_v)
            pltpu.sync_copy(mbuf_v, acc_sh.at[idx_v], add=True)
            return _

        lax.fori_loop(0, nchunk, body, 0)
        plsc.subcore_barrier()

        pltpu.sync_copy(acc_sh.at[pl.ds(sid * nrows, nrows)],
                        agg_hbm.at[pl.ds(p * _NHALF + sid * nrows, nrows),
                                   pl.ds(cid * 128, 128)])
        plsc.subcore_barrier()


def _sc_scatter_m(m, dst0, dst1):
    zero_c = jnp.zeros((_NHALF // 16, 128), jnp.float32)
    kfn = pl.kernel(
        _scatter_m_kernel,
        out_type=jax.ShapeDtypeStruct((_N_PAD, _C), jnp.float32),
        mesh=_sc_mesh(),
        scratch_types=[
            pltpu.VMEM((_SCHUNK,), jnp.int32),
            pltpu.VMEM((_SCHUNK, 128), jnp.float32),
            pltpu.VMEM_SHARED((_NHALF + 8, 128), jnp.float32),
            pltpu.SemaphoreType.DMA,
        ],
    )
    return kfn(m, dst0, dst1, zero_c)


def _scatter_wv_kernel(wv_hbm, dst_hbm, zero_p_hbm, dp_hbm,
                       widx_v, wvbuf_v, accp_sh, sem):
    cid = lax.axis_index("c")
    sid = lax.axis_index("s")
    nrows = _N_PAD // 16
    pltpu.sync_copy(zero_p_hbm, accp_sh.at[pl.ds(sid * nrows, nrows)])
    plsc.subcore_barrier()

    we0 = cid * (_E_PAD // 2) + sid * (_E_PAD // 32)
    pltpu.sync_copy(dst_hbm.at[pl.ds(we0, _E_PAD // 32)], widx_v)
    pltpu.sync_copy(wv_hbm.at[pl.ds(we0, _E_PAD // 32)], wvbuf_v)
    pltpu.sync_copy(wvbuf_v, accp_sh.at[widx_v], add=True)
    plsc.subcore_barrier()

    pltpu.sync_copy(accp_sh.at[pl.ds(sid * nrows, nrows)],
                    dp_hbm.at[cid, pl.ds(sid * nrows, nrows)])


def _sc_scatter_wv(wv, dst):
    zero_p = jnp.zeros((_N_PAD // 16, 8), jnp.float32)
    kfn = pl.kernel(
        _scatter_wv_kernel,
        out_type=jax.ShapeDtypeStruct((2, _N_PAD, 8), jnp.float32),
        mesh=_sc_mesh(),
        scratch_types=[
            pltpu.VMEM((_E_PAD // 32,), jnp.int32),
            pltpu.VMEM((_E_PAD // 32, 8), jnp.float32),
            pltpu.VMEM_SHARED((_N_PAD, 8), jnp.float32),
            pltpu.SemaphoreType.DMA,
        ],
        compiler_params=_SC_LINEAR,
    )
    return kfn(wv, dst, zero_p)


def _sc_gather(ps, pd, pos_pad, src, dst):
    asrc, adst = _sc_gather_feat(ps, pd, src, dst)
    psrc, pdst = _sc_gather_pos(pos_pad, src, dst)
    return asrc, adst, psrc, pdst


def _sc_scatter(m, wv, dst0, dst1, dst):
    agg = _sc_scatter_m(m, dst0, dst1)
    dp = _sc_scatter_wv(wv, dst)
    return agg, dp


def kernel(node_attrs, positions, edge_attrs, edge_index, params):
    src = edge_index[0].astype(jnp.int32)
    dst = edge_index[1].astype(jnp.int32)
    src_p = jnp.concatenate(
        [src, jnp.zeros((_E_PAD - _E,), jnp.int32)])
    dst_g = jnp.concatenate(
        [dst, jnp.zeros((_E_PAD - _E,), jnp.int32)])
    dst_s = jnp.concatenate(
        [dst, jnp.full((_E_PAD - _E,), _N, jnp.int32)])
    dst_s0 = jnp.where(dst_s < _NHALF, dst_s, _NHALF)
    dst_s1 = jnp.where(dst_s >= _NHALF, dst_s - _NHALF, _NHALF)

    pos_pad = jnp.zeros((_N, 8), jnp.float32).at[:, :3].set(positions)
    ea_pad = jnp.zeros((_E_PAD, edge_attrs.shape[1]), jnp.float32
                       ).at[:_E].set(edge_attrs)

    blocks = params['blocks']

    def blk_w(i):
        bp = blocks[i]
        w1 = bp['phi_e'][0]['W']
        return dict(
            ws=w1[0:_C], wd=w1[_C:2 * _C], wr=w1[2 * _C:2 * _C + _NUM_RBF],
            we=w1[2 * _C + _NUM_RBF:],
            b1=bp['phi_e'][0]['b'][None, :],
            w2=bp['phi_e'][1]['W'], b2=bp['phi_e'][1]['b'][None, :],
            w1h=bp['phi_h'][0]['W'][0:_C], w1a=bp['phi_h'][0]['W'][_C:],
            b1h=bp['phi_h'][0]['b'][None, :],
            w2h=bp['phi_h'][1]['W'], b2h=bp['phi_h'][1]['b'][None, :],
            wx1=bp['phi_x'][0]['W'], bx1=bp['phi_x'][0]['b'][None, :],
            wx2r=jnp.tile(bp['phi_x'][1]['W'], (1, 8)),
            bx2r=jnp.tile(bp['phi_x'][1]['b'][None, :], (1, 8)),
            gamma=params['ssf'][i]['gamma'], beta=params['ssf'][i]['beta'],
        )

    bw = [blk_w(i) for i in range(_NUM_BLOCKS)]

    en = params['embed_nodes']
    h, ps, pd = _embed_nodes(node_attrs, en[0]['W'], en[0]['b'][None, :],
                             en[1]['W'], en[1]['b'][None, :],
                             bw[0]['ws'], bw[0]['wd'])
    ee = params['embed_edges']
    ef = _embed_edges(ea_pad, ee[0]['W'], ee[0]['b'][None, :],
                      ee[1]['W'], ee[1]['b'][None, :])

    pos = pos_pad
    for i in range(_NUM_BLOCKS):
        w = bw[i]
        asrc, adst, psrc, pdst = _sc_gather(ps, pd, pos, src_p, dst_g)
        m, wv = _edge_mlp(asrc, adst, ef, psrc, pdst,
                          w['we'], w['wr'], w['b1'], w['w2'], w['b2'],
                          w['wx1'], w['bx1'], w['wx2r'], w['bx2r'])
        agg, dp = _sc_scatter(m, wv, dst_s0, dst_s1, dst_s)
        nxt = bw[i + 1] if i + 1 < _NUM_BLOCKS else None
        h, pos, ps, pd = _node_update(
            h, agg[:_N], pos, dp[0, :_N], dp[1, :_N],
            w['w1h'], w['w1a'], w['b1h'], w['w2h'], w['b2h'],
            w['gamma'], w['beta'],
            ws=nxt['ws'] if nxt else None, wd=nxt['wd'] if nxt else None)

    return h, pos[:, :3]
